# speedup vs baseline: 1859.8346x; 1859.8346x over previous
import jax
import jax.numpy as jnp
from jax.experimental import pallas as pl


def _gcn_stack_kernel(rpa_t_ref, x_ref, Wi_ref, W0_ref, W1_ref, Wo_ref,
                      bi_ref, b0_ref, b1_ref, bo_ref, out_ref):
    n = rpa_t_ref.shape[1]
    rpa_t = rpa_t_ref[0]
    row = jax.lax.broadcasted_iota(jnp.int32, (n, n), 0)
    col = jax.lax.broadcasted_iota(jnp.int32, (n, n), 1)
    at = jnp.where((rpa_t != 0) & (row != col), 1.0, 0.0).astype(jnp.float32)
    deg = jnp.sum(at, axis=1, keepdims=True) + 1.0
    dis = jax.lax.rsqrt(deg)

    def layer(xin, w_ref, b_ref):
        h = jnp.dot(xin, w_ref[...], preferred_element_type=jnp.float32)
        y = dis * h
        z = jnp.dot(at, y, preferred_element_type=jnp.float32) + y
        return jnp.maximum(dis * z + b_ref[...], 0.0)

    r1 = layer(x_ref[0], Wi_ref, bi_ref)
    r2 = layer(r1, W0_ref, b0_ref)
    r3 = layer(r2, W1_ref, b1_ref)
    r4 = layer(r3, Wo_ref, bo_ref)
    out_ref[0] = (r1 + r2 + r3 + r4) * 0.25


def kernel(aa_rep, rpa, W_in, b_in, W_h0, b_h0, W_h1, b_h1, W_out, b_out):
    B, N, Fi = aa_rep.shape
    Fo = W_out.shape[1]
    rpa_t = jnp.swapaxes(rpa, 1, 2)
    biases = [b.reshape(1, -1) for b in (b_in, b_h0, b_h1, b_out)]
    weights = [W_in, W_h0, W_h1, W_out]

    def rep_spec(shape):
        return pl.BlockSpec(shape, lambda i: (0,) * len(shape))

    out = pl.pallas_call(
        _gcn_stack_kernel,
        grid=(B,),
        in_specs=[
            pl.BlockSpec((1, N, N), lambda i: (i, 0, 0)),
            pl.BlockSpec((1, N, Fi), lambda i: (i, 0, 0)),
            *[rep_spec(w.shape) for w in weights],
            *[rep_spec(b.shape) for b in biases],
        ],
        out_specs=pl.BlockSpec((1, N, Fo), lambda i: (i, 0, 0)),
        out_shape=jax.ShapeDtypeStruct((B, N, Fo), jnp.float32),
    )(rpa_t, aa_rep, *weights, *biases)
    return out


# speedup vs baseline: 2867.2589x; 1.5417x over previous
import jax
import jax.numpy as jnp
from jax.experimental import pallas as pl


def _gcn_stack_kernel(rpa_ref, xt_ref, Wi_ref, W0_ref, W1_ref, Wo_ref,
                      bi_ref, b0_ref, b1_ref, bo_ref, out_ref):
    n = rpa_ref.shape[1]
    rpa = rpa_ref[0]
    row = jax.lax.broadcasted_iota(jnp.int32, (n, n), 0)
    col = jax.lax.broadcasted_iota(jnp.int32, (n, n), 1)
    a = jnp.where((rpa != 0) & (row != col), 1.0, 0.0).astype(jnp.bfloat16)
    ones = jnp.ones((1, n), jnp.bfloat16)
    deg = jnp.dot(ones, a, preferred_element_type=jnp.float32) + 1.0
    dis = jax.lax.rsqrt(deg)

    def layer(xt, wt_ref, b_ref):
        ht = jnp.dot(wt_ref[...], xt, preferred_element_type=jnp.float32)
        y = dis * ht
        z = jnp.dot(y.astype(jnp.bfloat16), a,
                    preferred_element_type=jnp.float32) + y
        return jnp.maximum(dis * z + b_ref[...], 0.0)

    r1 = layer(xt_ref[0], Wi_ref, bi_ref)
    r2 = layer(r1, W0_ref, b0_ref)
    r3 = layer(r2, W1_ref, b1_ref)
    r4 = layer(r3, Wo_ref, bo_ref)
    out_ref[0] = (r1 + r2 + r3 + r4) * 0.25


def kernel(aa_rep, rpa, W_in, b_in, W_h0, b_h0, W_h1, b_h1, W_out, b_out):
    B, N, Fi = aa_rep.shape
    Fo = W_out.shape[1]
    xt = jnp.swapaxes(aa_rep, 1, 2)
    wts = [W.T for W in (W_in, W_h0, W_h1, W_out)]
    bcols = [b.reshape(-1, 1) for b in (b_in, b_h0, b_h1, b_out)]

    def rep_spec(shape):
        return pl.BlockSpec(shape, lambda i: (0,) * len(shape))

    out_t = pl.pallas_call(
        _gcn_stack_kernel,
        grid=(B,),
        in_specs=[
            pl.BlockSpec((1, N, N), lambda i: (i, 0, 0)),
            pl.BlockSpec((1, Fi, N), lambda i: (i, 0, 0)),
            *[rep_spec(w.shape) for w in wts],
            *[rep_spec(b.shape) for b in bcols],
        ],
        out_specs=pl.BlockSpec((1, Fo, N), lambda i: (i, 0, 0)),
        out_shape=jax.ShapeDtypeStruct((B, Fo, N), jnp.float32),
    )(rpa, xt, *wts, *bcols)
    return jnp.swapaxes(out_t, 1, 2)


# speedup vs baseline: 3569.6226x; 1.2450x over previous
import jax
import jax.numpy as jnp
from jax.experimental import pallas as pl


def _gcn_stack_kernel(rpa_ref, x_ref, Wi_ref, W0_ref, W1_ref, Wo_ref,
                      bi_ref, b0_ref, b1_ref, bo_ref, out_ref):
    n = rpa_ref.shape[1]
    rpa = rpa_ref[0]
    row = jax.lax.broadcasted_iota(jnp.int32, (n, n), 0)
    col = jax.lax.broadcasted_iota(jnp.int32, (n, n), 1)
    a = jnp.where((rpa != 0) & (row != col), 1.0, 0.0).astype(jnp.bfloat16)
    ones = jnp.ones((1, n), jnp.bfloat16)
    deg = jnp.dot(ones, a, preferred_element_type=jnp.float32) + 1.0
    dis = jax.lax.rsqrt(deg)

    def layer(xt, wt_ref, b_ref):
        ht = jnp.dot(wt_ref[...], xt.astype(jnp.bfloat16),
                     preferred_element_type=jnp.float32)
        y = dis * ht
        z = jnp.dot(y.astype(jnp.bfloat16), a,
                    preferred_element_type=jnp.float32) + y
        return jnp.maximum(dis * z + b_ref[...], 0.0)

    xt = jnp.transpose(x_ref[0])
    r1 = layer(xt, Wi_ref, bi_ref)
    r2 = layer(r1, W0_ref, b0_ref)
    r3 = layer(r2, W1_ref, b1_ref)
    r4 = layer(r3, Wo_ref, bo_ref)
    out_ref[0] = jnp.transpose((r1 + r2 + r3 + r4) * 0.25)


def kernel(aa_rep, rpa, W_in, b_in, W_h0, b_h0, W_h1, b_h1, W_out, b_out):
    B, N, Fi = aa_rep.shape
    Fo = W_out.shape[1]
    wts = [W.T.astype(jnp.bfloat16) for W in (W_in, W_h0, W_h1, W_out)]
    bcols = [b.reshape(-1, 1) for b in (b_in, b_h0, b_h1, b_out)]

    def rep_spec(shape):
        return pl.BlockSpec(shape, lambda i: (0,) * len(shape))

    return pl.pallas_call(
        _gcn_stack_kernel,
        grid=(B,),
        in_specs=[
            pl.BlockSpec((1, N, N), lambda i: (i, 0, 0)),
            pl.BlockSpec((1, N, Fi), lambda i: (i, 0, 0)),
            *[rep_spec(w.shape) for w in wts],
            *[rep_spec(b.shape) for b in bcols],
        ],
        out_specs=pl.BlockSpec((1, N, Fo), lambda i: (i, 0, 0)),
        out_shape=jax.ShapeDtypeStruct((B, N, Fo), jnp.float32),
    )(rpa, aa_rep, *wts, *bcols)


# speedup vs baseline: 8072.3103x; 2.2614x over previous
import jax
import jax.numpy as jnp
from jax.experimental import pallas as pl


def _gcn_stack_kernel(rpa_ref, x_ref, Wi_ref, W0_ref, W1_ref, Wo_ref,
                      bi_ref, b0_ref, b1_ref, bo_ref, out_ref):
    n = rpa_ref.shape[1]
    rpa = rpa_ref[0]
    row = jax.lax.broadcasted_iota(jnp.int32, (n, n), 0)
    col = jax.lax.broadcasted_iota(jnp.int32, (n, n), 1)
    a = jnp.where((rpa != 0) & (row != col), 1.0, 0.0).astype(jnp.bfloat16)
    ones = jnp.ones((1, n), jnp.bfloat16)
    deg = jnp.dot(ones, a, preferred_element_type=jnp.float32) + 1.0
    dis = jax.lax.rsqrt(deg)

    def layer(xt, w_ref, b_ref):
        ht = jax.lax.dot_general(
            w_ref[...].astype(jnp.bfloat16), xt.astype(jnp.bfloat16),
            (((0,), (0,)), ((), ())), preferred_element_type=jnp.float32)
        y = dis * ht
        z = jnp.dot(y.astype(jnp.bfloat16), a,
                    preferred_element_type=jnp.float32) + y
        return jnp.maximum(dis * z + jnp.transpose(b_ref[...]), 0.0)

    xt = jnp.transpose(x_ref[0])
    r1 = layer(xt, Wi_ref, bi_ref)
    r2 = layer(r1, W0_ref, b0_ref)
    r3 = layer(r2, W1_ref, b1_ref)
    r4 = layer(r3, Wo_ref, bo_ref)
    out_ref[0] = jnp.transpose((r1 + r2 + r3 + r4) * 0.25)


def kernel(aa_rep, rpa, W_in, b_in, W_h0, b_h0, W_h1, b_h1, W_out, b_out):
    B, N, Fi = aa_rep.shape
    Fo = W_out.shape[1]
    ws = [W_in, W_h0, W_h1, W_out]
    brows = [b.reshape(1, -1) for b in (b_in, b_h0, b_h1, b_out)]

    def rep_spec(shape):
        return pl.BlockSpec(shape, lambda i: (0,) * len(shape))

    return pl.pallas_call(
        _gcn_stack_kernel,
        grid=(B,),
        in_specs=[
            pl.BlockSpec((1, N, N), lambda i: (i, 0, 0)),
            pl.BlockSpec((1, N, Fi), lambda i: (i, 0, 0)),
            *[rep_spec(w.shape) for w in ws],
            *[rep_spec(b.shape) for b in brows],
        ],
        out_specs=pl.BlockSpec((1, N, Fo), lambda i: (i, 0, 0)),
        out_shape=jax.ShapeDtypeStruct((B, N, Fo), jnp.float32),
    )(rpa, aa_rep, *ws, *brows)
